# baseline (speedup 1.0000x reference)
import functools

import jax
import jax.numpy as jnp
from jax import lax
from jax.experimental import pallas as pl
from jax.experimental.pallas import tpu as pltpu
from jax.experimental.pallas import tpu_sc as plsc

_NC, _NS, _LANES = 2, 16, 16
_NW = _NC * _NS
_CH = 512
_NBUF = 4
_RCH = 128


def _sc_transpose(x_flat, B, L):
    bpw = B // _NW
    nrch = bpw // _RCH
    nl16 = (L + _LANES - 1) // _LANES

    mesh = plsc.VectorSubcoreMesh(core_axis_name="c", subcore_axis_name="s")

    @functools.partial(
        pl.kernel,
        out_type=jax.ShapeDtypeStruct((L, B), jnp.int32),
        mesh=mesh,
        scratch_types=[
            pltpu.VMEM((_RCH * L + _LANES,), jnp.int32),
            pltpu.VMEM((L, bpw), jnp.int32),
        ],
        compiler_params=pltpu.CompilerParams(
            use_tc_tiling_on_sc=False, needs_layout_passes=False),
    )
    def k(x_hbm, xT_hbm, in_v, col_v):
        wid = lax.axis_index("s") * _NC + lax.axis_index("c")
        base = wid * bpw
        lane = lax.iota(jnp.int32, _LANES)

        def chunk(c, carry):
            pltpu.sync_copy(x_hbm.at[pl.ds((base + c * _RCH) * L, _RCH * L)],
                            in_v.at[pl.ds(0, _RCH * L)])

            def row(b, _):
                col = jnp.full((_LANES,), c * _RCH + b, jnp.int32)
                for li in range(nl16):
                    vals = in_v[pl.ds(b * L + li * _LANES, _LANES)]
                    lrow = lane + (li * _LANES)
                    if (li + 1) * _LANES <= L:
                        plsc.store_scatter(col_v, [lrow, col], vals)
                    else:
                        plsc.store_scatter(col_v, [lrow, col], vals,
                                           mask=lrow < L)
                return _

            lax.fori_loop(0, _RCH, row, 0)
            return carry

        lax.fori_loop(0, nrch, chunk, 0)
        pltpu.sync_copy(col_v, xT_hbm.at[:, pl.ds(base, bpw)])

    return k(x_flat)


def _sc_col_sums(xT, emb_table):
    L, B = xT.shape
    _, DIM = emb_table.shape
    nch = B // _CH
    n_iter = (L + _NW - 1) // _NW

    mesh = plsc.VectorSubcoreMesh(core_axis_name="c", subcore_axis_name="s")

    @functools.partial(
        pl.kernel,
        out_type=jax.ShapeDtypeStruct((L, DIM), jnp.float32),
        mesh=mesh,
        scratch_types=[
            pltpu.VMEM((B,), jnp.int32),
            [pltpu.VMEM((_CH, DIM), jnp.float32) for _ in range(_NBUF)],
            pltpu.VMEM((DIM,), jnp.float32),
            [pltpu.SemaphoreType.DMA for _ in range(_NBUF)],
        ],
        compiler_params=pltpu.CompilerParams(use_tc_tiling_on_sc=False),
    )
    def k(xT_hbm, table_hbm, out_hbm, idx_v, accs, row_v, sems):
        wid = lax.axis_index("s") * _NC + lax.axis_index("c")

        def body_i(i, carry):
            l = i * _NW + wid

            @pl.when(l < L)
            def _():
                pltpu.sync_copy(xT_hbm.at[l], idx_v)
                for n in range(_NBUF):
                    pltpu.async_copy(
                        table_hbm.at[idx_v.at[pl.ds(n * _CH, _CH)]],
                        accs[n], sems[n])

                def grp(p, c0):
                    for n in range(_NBUF):
                        pltpu.make_async_copy(
                            table_hbm.at[idx_v.at[pl.ds(0, _CH)]],
                            accs[n], sems[n]).wait()
                        pltpu.async_copy(
                            table_hbm.at[idx_v.at[pl.ds((c0 + n) * _CH, _CH)]],
                            accs[n], sems[n], add=True)
                    return c0 + _NBUF

                lax.fori_loop(1, nch // _NBUF, grp, _NBUF)
                for n in range(_NBUF):
                    pltpu.make_async_copy(
                        table_hbm.at[idx_v.at[pl.ds(0, _CH)]],
                        accs[n], sems[n]).wait()

                def red(g, acc):
                    a0, a1 = acc
                    for n in range(_NBUF):
                        a0 = a0 + accs[n][g, pl.ds(0, _LANES)]
                        a1 = a1 + accs[n][g, pl.ds(_LANES, _LANES)]
                    return (a0, a1)

                z = jnp.zeros((_LANES,), jnp.float32)
                a0, a1 = lax.fori_loop(0, _CH, red, (z, z), unroll=4)
                row_v[pl.ds(0, _LANES)] = a0
                row_v[pl.ds(_LANES, _LANES)] = a1
                pltpu.sync_copy(row_v, out_hbm.at[l])

            return carry

        lax.fori_loop(0, n_iter, body_i, 0)

    return k(xT, emb_table)


def _tc_finish(sums, W, b2d, n_total):
    def body(sums_ref, w_ref, b_ref, out_ref):
        m = sums_ref[...] * (1.0 / n_total)
        out_ref[...] = lax.dot_general(
            m, w_ref[...], (((1,), (1,)), ((), ())),
            preferred_element_type=jnp.float32) + b_ref[...]

    L, DIM = sums.shape
    return pl.pallas_call(
        body,
        out_shape=jax.ShapeDtypeStruct((L, DIM), jnp.float32),
    )(sums, W, b2d)


def kernel(x, emb_table, W, b):
    B, L = x.shape
    x_flat = jnp.maximum(jnp.reshape(x.astype(jnp.int32), (B * L,)), 0)
    xT = _sc_transpose(x_flat, B, L)
    sums = _sc_col_sums(xT, emb_table)
    return _tc_finish(sums, W, b.reshape(1, -1), B)


# speedup vs baseline: 1.0218x; 1.0218x over previous
import functools

import jax
import jax.numpy as jnp
from jax import lax
from jax.experimental import pallas as pl
from jax.experimental.pallas import tpu as pltpu
from jax.experimental.pallas import tpu_sc as plsc

_NC, _NS, _LANES = 2, 16, 16
_NW = _NC * _NS
_CH = 128
_GRP = 4
_NBUF = 4


def _sc_col_sums(x4, emb_table, L, B):
    _, DIM = emb_table.shape
    nch = B // _CH
    n_iter = (L + _NW - 1) // _NW

    mesh = plsc.VectorSubcoreMesh(core_axis_name="c", subcore_axis_name="s")

    @functools.partial(
        pl.kernel,
        out_type=jax.ShapeDtypeStruct((L, DIM), jnp.float32),
        mesh=mesh,
        scratch_types=[
            pltpu.VMEM((2, nch, _CH), jnp.int32),
            pltpu.VMEM((B,), jnp.int32),
            [pltpu.VMEM((_GRP * _CH, DIM), jnp.float32) for _ in range(_NBUF)],
            pltpu.VMEM((DIM,), jnp.float32),
            pltpu.SemaphoreType.DMA,
            [pltpu.SemaphoreType.DMA for _ in range(_NBUF)],
        ],
        compiler_params=pltpu.CompilerParams(use_tc_tiling_on_sc=False),
    )
    def k(x4_hbm, table_hbm, out_hbm, idx_v, idx1, accs, row_v, sem_idx, sems):
        wid = lax.axis_index("s") * _NC + lax.axis_index("c")

        def fetch_idx(l, slot):
            pltpu.async_copy(
                x4_hbm.at[l // 8, :, l % 8, :], idx_v.at[slot], sem_idx)

        l0 = wid

        @pl.when(l0 < L)
        def _():
            fetch_idx(l0, 0)

        def body_i(i, carry):
            l = i * _NW + wid
            slot = i % 2

            @pl.when(l < L)
            def _():
                pltpu.make_async_copy(
                    x4_hbm.at[0, :, 0, :], idx_v.at[0], sem_idx).wait()
                l_next = l + _NW

                @pl.when(l_next < L)
                def _():
                    fetch_idx(l_next, 1 - slot)

                def flat(q, carry):
                    for u in range(_CH // _LANES):
                        idx1[pl.ds(q * _CH + u * _LANES, _LANES)] = (
                            idx_v[slot, q, pl.ds(u * _LANES, _LANES)])
                    return carry

                lax.fori_loop(0, nch, flat, 0)

                gch = _GRP * _CH
                for n in range(_NBUF):
                    pltpu.async_copy(
                        table_hbm.at[idx1.at[pl.ds(n * gch, gch)]],
                        accs[n], sems[n])

                def grp(p, c0):
                    for n in range(_NBUF):
                        pltpu.make_async_copy(
                            table_hbm.at[idx1.at[pl.ds(0, gch)]],
                            accs[n], sems[n]).wait()
                        pltpu.async_copy(
                            table_hbm.at[idx1.at[pl.ds((c0 + n) * gch, gch)]],
                            accs[n], sems[n], add=True)
                    return c0 + _NBUF

                lax.fori_loop(1, B // (gch * _NBUF), grp, _NBUF)

                z = jnp.zeros((_LANES,), jnp.float32)
                a0, a1 = z, z
                for n in range(_NBUF):
                    pltpu.make_async_copy(
                        table_hbm.at[idx1.at[pl.ds(0, gch)]],
                        accs[n], sems[n]).wait()

                    def red(g, acc, _n=n):
                        b0, b1 = acc
                        return (b0 + accs[_n][g, pl.ds(0, _LANES)],
                                b1 + accs[_n][g, pl.ds(_LANES, _LANES)])

                    a0, a1 = lax.fori_loop(0, gch, red, (a0, a1), unroll=8)
                row_v[pl.ds(0, _LANES)] = a0
                row_v[pl.ds(_LANES, _LANES)] = a1
                pltpu.sync_copy(row_v, out_hbm.at[l])

            return carry

        lax.fori_loop(0, n_iter, body_i, 0)

    return k(x4, emb_table)


def _tc_finish(sums, W, b2d, n_total):
    def body(sums_ref, w_ref, b_ref, out_ref):
        m = sums_ref[...] * (1.0 / n_total)
        out_ref[...] = lax.dot_general(
            m, w_ref[...], (((1,), (1,)), ((), ())),
            preferred_element_type=jnp.float32) + b_ref[...]

    L, DIM = sums.shape
    return pl.pallas_call(
        body,
        out_shape=jax.ShapeDtypeStruct((L, DIM), jnp.float32),
    )(sums, W, b2d)


def kernel(x, emb_table, W, b):
    B, L = x.shape
    xT = jnp.transpose(x.astype(jnp.int32))
    x4 = jnp.transpose(
        jnp.reshape(xT, (L // 8, 8, B // 128, 128)), (0, 2, 1, 3))
    V, DIM = emb_table.shape
    t128 = lax.optimization_barrier(
        jnp.reshape(emb_table, (V * DIM // 128, 128)))
    table_lin = jnp.reshape(t128, (V, DIM))
    sums = _sc_col_sums(x4, table_lin, L, B)
    return _tc_finish(sums, W, b.reshape(1, -1), B)
